# baseline (device time: 46624 ns/iter reference)
import jax
import jax.numpy as jnp
from jax import lax
from jax.experimental import pallas as pl
from jax.experimental.pallas import tpu as pltpu

N_DEV = 4


def kernel(x, w_mat):
    m, k_shard = x.shape
    _, n = w_mat.shape
    m_per = m // N_DEV

    def body(x_ref, w_ref, out_ref, send_buf, recv_buf, send_sems, recv_sems):
        my = lax.axis_index("i")
        left = lax.rem(my + N_DEV - 1, N_DEV)
        right = lax.rem(my + 1, N_DEV)

        barrier_sem = pltpu.get_barrier_semaphore()
        for nbr in (left, right):
            pl.semaphore_signal(
                barrier_sem, inc=1,
                device_id=(nbr,), device_id_type=pl.DeviceIdType.MESH,
            )
        pl.semaphore_wait(barrier_sem, 2)

        def partial_chunk(c):
            xs = x_ref[pl.ds(c * m_per, m_per), :]
            return jnp.dot(xs, w_ref[:, :], preferred_element_type=jnp.float32)

        send_buf[0, :, :] = partial_chunk(lax.rem(my + N_DEV - 1, N_DEV))

        for h in range(N_DEV - 1):
            rdma = pltpu.make_async_remote_copy(
                src_ref=send_buf.at[h],
                dst_ref=recv_buf.at[h],
                send_sem=send_sems.at[h],
                recv_sem=recv_sems.at[h],
                device_id=(right,),
                device_id_type=pl.DeviceIdType.MESH,
            )
            rdma.start()
            rdma.wait()

            c_recv = lax.rem(my + N_DEV - 2 - h, N_DEV)
            if h < N_DEV - 2:
                send_buf[h + 1, :, :] = recv_buf[h, :, :] + partial_chunk(c_recv)
            else:
                acc = recv_buf[h, :, :] + partial_chunk(c_recv)
                out_ref[:, :] = jnp.maximum(acc, 0.0)

    return pl.pallas_call(
        body,
        out_shape=jax.ShapeDtypeStruct((m_per, n), jnp.float32),
        in_specs=[
            pl.BlockSpec(memory_space=pltpu.VMEM),
            pl.BlockSpec(memory_space=pltpu.VMEM),
        ],
        out_specs=pl.BlockSpec(memory_space=pltpu.VMEM),
        scratch_shapes=[
            pltpu.VMEM((N_DEV - 1, m_per, n), jnp.float32),
            pltpu.VMEM((N_DEV - 1, m_per, n), jnp.float32),
            pltpu.SemaphoreType.DMA((N_DEV - 1,)),
            pltpu.SemaphoreType.DMA((N_DEV - 1,)),
        ],
        compiler_params=pltpu.CompilerParams(collective_id=0),
    )(x, w_mat)


# device time: 29269 ns/iter; 1.5929x vs baseline; 1.5929x over previous
import jax
import jax.numpy as jnp
from jax import lax
from jax.experimental import pallas as pl
from jax.experimental.pallas import tpu as pltpu

N_DEV = 4
N_HOP = N_DEV - 1


def kernel(x, w_mat):
    m, k_shard = x.shape
    _, n = w_mat.shape
    m_per = m // N_DEV
    nh = n // 2

    def body(x_ref, w_ref, out_ref,
             sb_r, sb_l, rb_r, rb_l,
             ss_r, sr_r, ss_l, sr_l):
        my = lax.axis_index("i")
        left = lax.rem(my + N_DEV - 1, N_DEV)
        right = lax.rem(my + 1, N_DEV)

        barrier_sem = pltpu.get_barrier_semaphore()
        for nbr in (left, right):
            pl.semaphore_signal(
                barrier_sem, inc=1,
                device_id=(nbr,), device_id_type=pl.DeviceIdType.MESH,
            )
        pl.semaphore_wait(barrier_sem, 2)

        def pchunk(c, col0):
            xs = x_ref[pl.ds(c * m_per, m_per), :]
            return jnp.dot(xs, w_ref[:, col0:col0 + nh],
                           preferred_element_type=jnp.float32)

        def mk(sb, rb, ss, sr, h, dev):
            return pltpu.make_async_remote_copy(
                src_ref=sb.at[h], dst_ref=rb.at[h],
                send_sem=ss.at[h], recv_sem=sr.at[h],
                device_id=(dev,), device_id_type=pl.DeviceIdType.MESH,
            )

        rdma_r = [mk(sb_r, rb_r, ss_r, sr_r, h, right) for h in range(N_HOP)]
        rdma_l = [mk(sb_l, rb_l, ss_l, sr_l, h, left) for h in range(N_HOP)]

        sb_r[0, :, :] = pchunk(lax.rem(my + 3, N_DEV), 0)
        rdma_r[0].start()
        sb_l[0, :, :] = pchunk(lax.rem(my + 1, N_DEV), nh)
        rdma_l[0].start()

        sb_r[1, :, :] = pchunk(lax.rem(my + 2, N_DEV), 0)
        sb_l[1, :, :] = pchunk(lax.rem(my + 2, N_DEV), nh)
        sb_r[2, :, :] = pchunk(lax.rem(my + 1, N_DEV), 0)
        sb_l[2, :, :] = pchunk(lax.rem(my + 3, N_DEV), nh)
        own_a = pchunk(my, 0)
        own_b = pchunk(my, nh)

        for h in range(N_HOP - 1):
            rdma_r[h].wait_recv()
            sb_r[h + 1, :, :] = sb_r[h + 1, :, :] + rb_r[h, :, :]
            rdma_r[h + 1].start()
            rdma_l[h].wait_recv()
            sb_l[h + 1, :, :] = sb_l[h + 1, :, :] + rb_l[h, :, :]
            rdma_l[h + 1].start()

        rdma_r[N_HOP - 1].wait_recv()
        out_ref[:, 0:nh] = jnp.maximum(rb_r[N_HOP - 1, :, :] + own_a, 0.0)
        rdma_l[N_HOP - 1].wait_recv()
        out_ref[:, nh:n] = jnp.maximum(rb_l[N_HOP - 1, :, :] + own_b, 0.0)

        for h in range(N_HOP):
            rdma_r[h].wait_send()
            rdma_l[h].wait_send()

    return pl.pallas_call(
        body,
        out_shape=jax.ShapeDtypeStruct((m_per, n), jnp.float32),
        in_specs=[
            pl.BlockSpec(memory_space=pltpu.VMEM),
            pl.BlockSpec(memory_space=pltpu.VMEM),
        ],
        out_specs=pl.BlockSpec(memory_space=pltpu.VMEM),
        scratch_shapes=[
            pltpu.VMEM((N_HOP, m_per, nh), jnp.float32),
            pltpu.VMEM((N_HOP, m_per, nh), jnp.float32),
            pltpu.VMEM((N_HOP, m_per, nh), jnp.float32),
            pltpu.VMEM((N_HOP, m_per, nh), jnp.float32),
            pltpu.SemaphoreType.DMA((N_HOP,)),
            pltpu.SemaphoreType.DMA((N_HOP,)),
            pltpu.SemaphoreType.DMA((N_HOP,)),
            pltpu.SemaphoreType.DMA((N_HOP,)),
        ],
        compiler_params=pltpu.CompilerParams(collective_id=0),
    )(x, w_mat)


# device time: 25869 ns/iter; 1.8023x vs baseline; 1.1314x over previous
import jax
import jax.numpy as jnp
from jax import lax
from jax.experimental import pallas as pl
from jax.experimental.pallas import tpu as pltpu

N_DEV = 4
N_HOP = N_DEV - 1
N_SEG = 4
_ORDER = (0, 2, 1, 3)


def kernel(x, w_mat):
    m, k_shard = x.shape
    _, n = w_mat.shape
    m_per = m // N_DEV
    nq = n // N_SEG

    def body(x_ref, w_ref, out_ref, sb, rb, ss, sr):
        my = lax.axis_index("i")
        left = lax.rem(my + N_DEV - 1, N_DEV)
        right = lax.rem(my + 1, N_DEV)

        barrier_sem = pltpu.get_barrier_semaphore()
        for nbr in (left, right):
            pl.semaphore_signal(
                barrier_sem, inc=1,
                device_id=(nbr,), device_id_type=pl.DeviceIdType.MESH,
            )
        pl.semaphore_wait(barrier_sem, 2)

        def goes_right(q):
            return q < N_SEG // 2

        def send_chunk(q, h):
            d = N_DEV - 1 - h if goes_right(q) else 1 + h
            return lax.rem(my + d, N_DEV)

        def recv_chunk(q, h):
            d = N_DEV - 2 - h if goes_right(q) else 2 + h
            return lax.rem(my + d, N_DEV)

        def pchunk(c, q):
            xs = x_ref[pl.ds(c * m_per, m_per), :]
            return jnp.dot(xs, w_ref[:, q * nq:(q + 1) * nq],
                           preferred_element_type=jnp.float32)

        def slot(q, h):
            return q * N_HOP + h

        def mk(q, h):
            dev = right if goes_right(q) else left
            return pltpu.make_async_remote_copy(
                src_ref=sb.at[slot(q, h)], dst_ref=rb.at[slot(q, h)],
                send_sem=ss.at[slot(q, h)], recv_sem=sr.at[slot(q, h)],
                device_id=(dev,), device_id_type=pl.DeviceIdType.MESH,
            )

        rdma = {(q, h): mk(q, h) for q in range(N_SEG) for h in range(N_HOP)}

        for q in _ORDER:
            sb[slot(q, 0), :, :] = pchunk(send_chunk(q, 0), q)
            rdma[(q, 0)].start()

        for h in range(N_HOP - 1):
            for q in _ORDER:
                sb[slot(q, h + 1), :, :] = pchunk(recv_chunk(q, h), q)
        own = {q: pchunk(my, q) for q in _ORDER}

        for h in range(N_HOP - 1):
            for q in _ORDER:
                rdma[(q, h)].wait_recv()
                sb[slot(q, h + 1), :, :] = (
                    sb[slot(q, h + 1), :, :] + rb[slot(q, h), :, :]
                )
                rdma[(q, h + 1)].start()

        for q in _ORDER:
            rdma[(q, N_HOP - 1)].wait_recv()
            out_ref[:, q * nq:(q + 1) * nq] = jnp.maximum(
                rb[slot(q, N_HOP - 1), :, :] + own[q], 0.0
            )

        for q in range(N_SEG):
            for h in range(N_HOP):
                rdma[(q, h)].wait_send()

    n_slots = N_SEG * N_HOP
    return pl.pallas_call(
        body,
        out_shape=jax.ShapeDtypeStruct((m_per, n), jnp.float32),
        in_specs=[
            pl.BlockSpec(memory_space=pltpu.VMEM),
            pl.BlockSpec(memory_space=pltpu.VMEM),
        ],
        out_specs=pl.BlockSpec(memory_space=pltpu.VMEM),
        scratch_shapes=[
            pltpu.VMEM((n_slots, m_per, nq), jnp.float32),
            pltpu.VMEM((n_slots, m_per, nq), jnp.float32),
            pltpu.SemaphoreType.DMA((n_slots,)),
            pltpu.SemaphoreType.DMA((n_slots,)),
        ],
        compiler_params=pltpu.CompilerParams(collective_id=0),
    )(x, w_mat)


# device time: 25668 ns/iter; 1.8164x vs baseline; 1.0078x over previous
import jax
import jax.numpy as jnp
from jax import lax
from jax.experimental import pallas as pl
from jax.experimental.pallas import tpu as pltpu

N_DEV = 4
N_HOP = N_DEV - 1
N_SEG = 8
_ORDER = (0, 4, 1, 5, 2, 6, 3, 7)


def kernel(x, w_mat):
    m, k_shard = x.shape
    _, n = w_mat.shape
    m_per = m // N_DEV
    nq = n // N_SEG

    def body(x_ref, w_ref, out_ref, sb, rb, ss, sr):
        my = lax.axis_index("i")
        left = lax.rem(my + N_DEV - 1, N_DEV)
        right = lax.rem(my + 1, N_DEV)

        def goes_right(q):
            return q < N_SEG // 2

        def send_chunk(q, h):
            d = N_DEV - 1 - h if goes_right(q) else 1 + h
            return lax.rem(my + d, N_DEV)

        def recv_chunk(q, h):
            d = N_DEV - 2 - h if goes_right(q) else 2 + h
            return lax.rem(my + d, N_DEV)

        def pchunk(c, q):
            xs = x_ref[pl.ds(c * m_per, m_per), :]
            return jnp.dot(xs, w_ref[:, q * nq:(q + 1) * nq],
                           preferred_element_type=jnp.float32)

        def slot(q, h):
            return q * N_HOP + h

        def mk(q, h):
            dev = right if goes_right(q) else left
            return pltpu.make_async_remote_copy(
                src_ref=sb.at[slot(q, h)], dst_ref=rb.at[slot(q, h)],
                send_sem=ss.at[slot(q, h)], recv_sem=sr.at[slot(q, h)],
                device_id=(dev,), device_id_type=pl.DeviceIdType.MESH,
            )

        rdma = {(q, h): mk(q, h) for q in range(N_SEG) for h in range(N_HOP)}

        barrier_sem = pltpu.get_barrier_semaphore()
        for nbr in (left, right):
            pl.semaphore_signal(
                barrier_sem, inc=1,
                device_id=(nbr,), device_id_type=pl.DeviceIdType.MESH,
            )
        for q in _ORDER:
            sb[slot(q, 0), :, :] = pchunk(send_chunk(q, 0), q)
        pl.semaphore_wait(barrier_sem, 2)
        for q in _ORDER:
            rdma[(q, 0)].start()

        for h in range(N_HOP - 1):
            for q in _ORDER:
                sb[slot(q, h + 1), :, :] = pchunk(recv_chunk(q, h), q)

        for q in _ORDER:
            rdma[(q, 0)].wait_recv()
            sb[slot(q, 1), :, :] = sb[slot(q, 1), :, :] + rb[slot(q, 0), :, :]
            rdma[(q, 1)].start()

        own = {q: pchunk(my, q) for q in _ORDER}

        for q in _ORDER:
            rdma[(q, 1)].wait_recv()
            sb[slot(q, 2), :, :] = sb[slot(q, 2), :, :] + rb[slot(q, 1), :, :]
            rdma[(q, 2)].start()

        for q in _ORDER:
            rdma[(q, N_HOP - 1)].wait_recv()
            out_ref[:, q * nq:(q + 1) * nq] = jnp.maximum(
                rb[slot(q, N_HOP - 1), :, :] + own[q], 0.0
            )

        for q in range(N_SEG):
            for h in range(N_HOP):
                rdma[(q, h)].wait_send()

    n_slots = N_SEG * N_HOP
    return pl.pallas_call(
        body,
        out_shape=jax.ShapeDtypeStruct((m_per, n), jnp.float32),
        in_specs=[
            pl.BlockSpec(memory_space=pltpu.VMEM),
            pl.BlockSpec(memory_space=pltpu.VMEM),
        ],
        out_specs=pl.BlockSpec(memory_space=pltpu.VMEM),
        scratch_shapes=[
            pltpu.VMEM((n_slots, m_per, nq), jnp.float32),
            pltpu.VMEM((n_slots, m_per, nq), jnp.float32),
            pltpu.SemaphoreType.DMA((n_slots,)),
            pltpu.SemaphoreType.DMA((n_slots,)),
        ],
        compiler_params=pltpu.CompilerParams(collective_id=0),
    )(x, w_mat)
